# baseline (speedup 1.0000x reference)
import functools

import jax
import jax.numpy as jnp
from jax import lax
from jax.experimental import pallas as pl
from jax.experimental.pallas import tpu as pltpu
from jax.experimental.pallas import tpu_sc as plsc

_EMB = 128
_BB = 32

_NC, _NS = 2, 16
_NW = _NC * _NS


_SLAB = 2


def _zeros_body(out_ref, buf, sem):
    wid = lax.axis_index("s") * _NC + lax.axis_index("c")
    bsz, r, emb = out_ref.shape
    per_w = bsz // _NW
    n_copies = per_w // _SLAB
    base = wid * per_w
    zeros16 = jnp.zeros((16,), jnp.float32)

    def _zb(i, _):
        bi = i // r
        ji = i % r
        for k in range(emb // 16):
            buf[bi, ji, pl.ds(k * 16, 16)] = zeros16
        return 0

    lax.fori_loop(0, _SLAB * r, _zb, 0)
    handles = [
        pltpu.async_copy(buf, out_ref.at[pl.ds(base + t * _SLAB, _SLAB)], sem)
        for t in range(n_copies)
    ]
    for h in handles:
        h.wait()


def _tc_body(row_in_ref, rand_ref, caps_ref, w_ref, b_ref, row_out_ref,
             col_ref, caps_out_ref):
    del row_in_ref, row_out_ref
    bb, c = rand_ref.shape
    k_sub = lax.broadcasted_iota(jnp.int32, (c, c), 0)
    j_lane = lax.broadcasted_iota(jnp.int32, (c, c), 1)
    tri = k_sub < j_lane
    n_sub = k_sub
    r_all = rand_ref[...]
    rt_all = jnp.transpose(r_all)
    for i in range(bb):
        rj = r_all[i:i + 1, :]
        rk = rt_all[:, i:i + 1]
        before = (rk < rj) | ((rk == rj) & tri)
        rank = jnp.sum(before.astype(jnp.int32), axis=0, keepdims=True)
        col_ref[i] = (n_sub == rank).astype(jnp.float32)
    acc = lax.dot_general(
        caps_ref[...], w_ref[...], (((1,), (1,)), ((), ())),
        preferred_element_type=jnp.float32,
        precision=lax.Precision.HIGHEST,
    )
    caps_out_ref[...] = acc + b_ref[...]


def kernel(cost_matrix, node_capacities, W, b):
    bsz, r, c = cost_matrix.shape
    m = node_capacities.shape[1]
    rand = jax.random.uniform(jax.random.key(42), (bsz, c))
    b2 = b.reshape(1, r)

    mesh = plsc.VectorSubcoreMesh(core_axis_name="c", subcore_axis_name="s")
    sc_zeros = functools.partial(
        pl.kernel,
        mesh=mesh,
        out_type=jax.ShapeDtypeStruct((bsz, r, _EMB), jnp.float32),
        scratch_types=[
            pltpu.VMEM((_SLAB, r, _EMB), jnp.float32),
            pltpu.SemaphoreType.DMA,
        ],
        compiler_params=pltpu.CompilerParams(use_tc_tiling_on_sc=True),
    )(_zeros_body)
    row_sc = sc_zeros()

    grid = bsz // _BB
    row_emb, col_emb, caps_out = pl.pallas_call(
        _tc_body,
        grid=(grid,),
        in_specs=[
            pl.BlockSpec(memory_space=pl.ANY),
            pl.BlockSpec((_BB, c), lambda i: (i, 0)),
            pl.BlockSpec((_BB, m), lambda i: (i, 0)),
            pl.BlockSpec((r, m), lambda i: (0, 0)),
            pl.BlockSpec((1, r), lambda i: (0, 0)),
        ],
        out_specs=[
            pl.BlockSpec(memory_space=pl.ANY),
            pl.BlockSpec((_BB, c, _EMB), lambda i: (i, 0, 0)),
            pl.BlockSpec((_BB, r), lambda i: (i, 0)),
        ],
        out_shape=[
            jax.ShapeDtypeStruct((bsz, r, _EMB), jnp.float32),
            jax.ShapeDtypeStruct((bsz, c, _EMB), cost_matrix.dtype),
            jax.ShapeDtypeStruct((bsz, r), jnp.float32),
        ],
        input_output_aliases={0: 0},
    )(row_sc, rand, node_capacities, W, b2)
    return (row_emb, col_emb, cost_matrix, caps_out)


# speedup vs baseline: 1.1190x; 1.1190x over previous
import functools

import jax
import jax.numpy as jnp
from jax import lax
from jax.experimental import pallas as pl
from jax.experimental.pallas import tpu as pltpu
from jax.experimental.pallas import tpu_sc as plsc

_EMB = 128
_BB = 32

_NC, _NS = 2, 16
_NW = _NC * _NS


_SLAB = 2


def _zeros_body(out_ref, buf, sem):
    wid = lax.axis_index("s") * _NC + lax.axis_index("c")
    bsz, r, emb = out_ref.shape
    per_w = bsz // _NW
    n_copies = per_w // _SLAB
    base = wid * per_w
    zeros16 = jnp.zeros((16,), jnp.float32)

    def _zb(i, _):
        bi = i // r
        ji = i % r
        for k in range(emb // 16):
            buf[bi, ji, pl.ds(k * 16, 16)] = zeros16
        return 0

    lax.fori_loop(0, _SLAB * r, _zb, 0)
    handles = [
        pltpu.async_copy(buf, out_ref.at[pl.ds(base + t * _SLAB, _SLAB)], sem)
        for t in range(n_copies)
    ]
    for h in handles:
        h.wait()


def _tc_body(dmat_in_ref, rand_ref, caps_ref, w_ref, b_ref, dmat_out_ref,
             col_ref, caps_out_ref):
    del dmat_in_ref, dmat_out_ref
    bb, c = rand_ref.shape
    k_sub = lax.broadcasted_iota(jnp.int32, (c, c), 0)
    j_lane = lax.broadcasted_iota(jnp.int32, (c, c), 1)
    tri = k_sub < j_lane
    n_sub = k_sub
    r_all = rand_ref[...]
    rt_all = jnp.transpose(r_all)
    for i in range(bb):
        rj = r_all[i:i + 1, :]
        rk = rt_all[:, i:i + 1]
        before = (rk < rj) | ((rk == rj) & tri)
        rank = jnp.sum(before.astype(jnp.int32), axis=0, keepdims=True)
        col_ref[i] = (n_sub == rank).astype(jnp.float32)
    acc = lax.dot_general(
        caps_ref[...], w_ref[...], (((1,), (1,)), ((), ())),
        preferred_element_type=jnp.float32,
        precision=lax.Precision.HIGHEST,
    )
    caps_out_ref[...] = acc + b_ref[...]


def kernel(cost_matrix, node_capacities, W, b):
    bsz, r, c = cost_matrix.shape
    m = node_capacities.shape[1]
    rand = jax.random.uniform(jax.random.key(42), (bsz, c))
    b2 = b.reshape(1, r)

    mesh = plsc.VectorSubcoreMesh(core_axis_name="c", subcore_axis_name="s")
    sc_zeros = functools.partial(
        pl.kernel,
        mesh=mesh,
        out_type=jax.ShapeDtypeStruct((bsz, r, _EMB), jnp.float32),
        scratch_types=[
            pltpu.VMEM((_SLAB, r, _EMB), jnp.float32),
            pltpu.SemaphoreType.DMA,
        ],
        compiler_params=pltpu.CompilerParams(use_tc_tiling_on_sc=True),
    )(_zeros_body)
    row_emb = sc_zeros()

    grid = bsz // _BB
    dmat, col_emb, caps_out = pl.pallas_call(
        _tc_body,
        grid=(grid,),
        in_specs=[
            pl.BlockSpec(memory_space=pl.ANY),
            pl.BlockSpec((_BB, c), lambda i: (i, 0)),
            pl.BlockSpec((_BB, m), lambda i: (i, 0)),
            pl.BlockSpec((r, m), lambda i: (0, 0)),
            pl.BlockSpec((1, r), lambda i: (0, 0)),
        ],
        out_specs=[
            pl.BlockSpec(memory_space=pl.ANY),
            pl.BlockSpec((_BB, c, _EMB), lambda i: (i, 0, 0)),
            pl.BlockSpec((_BB, r), lambda i: (i, 0)),
        ],
        out_shape=[
            jax.ShapeDtypeStruct((bsz, r, c), cost_matrix.dtype),
            jax.ShapeDtypeStruct((bsz, c, _EMB), cost_matrix.dtype),
            jax.ShapeDtypeStruct((bsz, r), jnp.float32),
        ],
        input_output_aliases={0: 0},
    )(cost_matrix, rand, node_capacities, W, b2)
    return (row_emb, col_emb, dmat, caps_out)


# speedup vs baseline: 1.2331x; 1.1019x over previous
import jax
import jax.numpy as jnp
from jax import lax
from jax.experimental import pallas as pl

_EMB = 128
_BB = 32


def _body(rand_ref, caps_ref, w_ref, b_ref, row_ref, col_ref, caps_out_ref):
    bb, c = rand_ref.shape
    k_sub = lax.broadcasted_iota(jnp.int32, (c, c), 0)
    j_lane = lax.broadcasted_iota(jnp.int32, (c, c), 1)
    tri = k_sub < j_lane
    n_sub = k_sub
    r_all = rand_ref[...]
    rt_all = jnp.transpose(r_all)
    for i in range(bb):
        rj = r_all[i:i + 1, :]
        rk = rt_all[:, i:i + 1]
        before = (rk < rj) | ((rk == rj) & tri)
        rank = jnp.sum(before.astype(jnp.int32), axis=0, keepdims=True)
        col_ref[i] = (n_sub == rank).astype(jnp.float32)
    row_ref[...] = jnp.zeros(row_ref.shape, row_ref.dtype)
    acc = lax.dot_general(
        caps_ref[...], w_ref[...], (((1,), (1,)), ((), ())),
        preferred_element_type=jnp.float32,
        precision=lax.Precision.HIGHEST,
    )
    caps_out_ref[...] = acc + b_ref[...]


def kernel(cost_matrix, node_capacities, W, b):
    bsz, r, c = cost_matrix.shape
    m = node_capacities.shape[1]
    with jax.ensure_compile_time_eval():
        rand = jax.random.uniform(jax.random.key(42), (bsz, c))
    b2 = b.reshape(1, r)
    grid = bsz // _BB
    row_emb, col_emb, caps_out = pl.pallas_call(
        _body,
        grid=(grid,),
        in_specs=[
            pl.BlockSpec((_BB, c), lambda i: (i, 0)),
            pl.BlockSpec((_BB, m), lambda i: (i, 0)),
            pl.BlockSpec((r, m), lambda i: (0, 0)),
            pl.BlockSpec((1, r), lambda i: (0, 0)),
        ],
        out_specs=[
            pl.BlockSpec((_BB, r, _EMB), lambda i: (i, 0, 0)),
            pl.BlockSpec((_BB, c, _EMB), lambda i: (i, 0, 0)),
            pl.BlockSpec((_BB, r), lambda i: (i, 0)),
        ],
        out_shape=[
            jax.ShapeDtypeStruct((bsz, r, _EMB), cost_matrix.dtype),
            jax.ShapeDtypeStruct((bsz, c, _EMB), cost_matrix.dtype),
            jax.ShapeDtypeStruct((bsz, r), jnp.float32),
        ],
    )(rand, node_capacities, W, b2)
    return (row_emb, col_emb, cost_matrix, caps_out)
